# baseline (speedup 1.0000x reference)
import functools

import jax
import jax.numpy as jnp
from jax import lax
from jax.experimental import pallas as pl
from jax.experimental.pallas import tpu as pltpu
from jax.experimental.pallas import tpu_sc as plsc

NC = 2
NS = 16
NW = NC * NS
D = 64
C = 128


@functools.partial(jax.jit, static_argnames=("b_per_w",))
def _emb_lookup(idx3, table, *, b_per_w):
    n_chunks = b_per_w // C
    B = NW * b_per_w

    mesh = plsc.VectorSubcoreMesh(core_axis_name="c", subcore_axis_name="s")

    NB = 2
    SUB = 4
    BC = SUB * C
    n_big = b_per_w // BC

    @functools.partial(
        pl.kernel,
        out_type=jax.ShapeDtypeStruct((B, D), jnp.float32),
        mesh=mesh,
        scratch_types=[
            pltpu.VMEM((n_chunks, C), jnp.int32),
            pltpu.VMEM((NB, BC, D), jnp.float32),
            pltpu.SemaphoreType.DMA,
            pltpu.SemaphoreType.DMA,
        ],
        compiler_params=pltpu.CompilerParams(use_tc_tiling_on_sc=False),
    )
    def emb(table_hbm, idx_hbm, out_hbm, idx_v, rows_v, sem_g, sem_s):
        wid = lax.axis_index("s") * NC + lax.axis_index("c")
        base = wid * b_per_w
        pltpu.sync_copy(idx_hbm.at[wid], idx_v)

        def gather_big(G, b):
            for u in range(SUB):
                pltpu.async_copy(
                    table_hbm.at[idx_v.at[G * SUB + u]],
                    rows_v.at[b, pl.ds(u * C, C)],
                    sem_g,
                )

        def wait_big(G, b):
            for u in range(SUB):
                pltpu.make_async_copy(
                    table_hbm.at[idx_v.at[G * SUB + u]],
                    rows_v.at[b, pl.ds(u * C, C)],
                    sem_g,
                ).wait()

        def store_desc(G, b):
            return pltpu.make_async_copy(
                rows_v.at[b], out_hbm.at[pl.ds(base + G * BC, BC)], sem_s
            )

        gather_big(0, 0)
        gather_big(1, 1)

        def group(grp, carry):
            for b in range(NB):
                G = grp * NB + b
                wait_big(G, b)
                store_desc(G, b).start()
                store_desc(G, b).wait()

                @pl.when(G + NB < n_big)
                def _():
                    gather_big(G + NB, b)

            return carry

        lax.fori_loop(0, n_big // NB, group, 0)

    return emb(table, idx3)


def kernel(x, Embeddings):
    B0, B1 = x.shape
    B = B0 * B1
    b_per_w = B // NW
    idx3 = x.astype(jnp.int32).reshape(NW, b_per_w // C, C)
    out = _emb_lookup(idx3, Embeddings, b_per_w=b_per_w)
    return out.reshape(B0, B1, D)
